# baseline (speedup 1.0000x reference)
import functools

import jax
import jax.numpy as jnp
from jax import lax
from jax.experimental import pallas as pl
from jax.experimental.pallas import tpu as pltpu
from jax.experimental.pallas import tpu_sc as plsc

N = 10002
N8 = 10008
E = 320064
D = 128
NT = 4
STEPS = 6

NTILES = 32
NPAD = 10112
RPT = NPAD // 16
CH = 96
NCHUNK = 108
EPT = NCHUNK * CH
EPAD = EPT * NTILES
NB = 3
IR = 6

_sc_mesh = plsc.VectorSubcoreMesh(core_axis_name="c", subcore_axis_name="s")


@functools.partial(
    pl.kernel,
    out_type=jax.ShapeDtypeStruct((2, NPAD, D), jnp.float32),
    mesh=_sc_mesh,
    scratch_types=[
        pltpu.VMEM((IR, CH), jnp.int32),
        pltpu.VMEM((IR, CH), jnp.int32),
        pltpu.VMEM((NB, CH, D), jnp.float32),
        pltpu.VMEM_SHARED((NPAD, D), jnp.float32),
    ] + [pltpu.SemaphoreType.DMA] * (2 * NB + IR),
)
def _edge_pass(lin_hbm, gidx_hbm, dst_hbm, zeros_hbm,
               out, idx_v, dst_v, rows, acc_sh, *sems):
    sem_g = sems[:NB]
    sem_s = sems[NB:2 * NB]
    sem_i = sems[2 * NB:]
    c = lax.axis_index("c")
    s = lax.axis_index("s")
    lo = s * RPT
    wid = c * 16 + s

    def load_idx(k, q, sync=False):
        if sync:
            pltpu.sync_copy(gidx_hbm.at[wid, k], idx_v.at[q])
            pltpu.sync_copy(dst_hbm.at[wid, k], dst_v.at[q])
        else:
            pltpu.async_copy(gidx_hbm.at[wid, k], idx_v.at[q], sem_i[q])
            pltpu.async_copy(dst_hbm.at[wid, k], dst_v.at[q], sem_i[q])

    def wait_idx(q):
        pltpu.make_async_copy(gidx_hbm.at[wid, 0], idx_v.at[q], sem_i[q]).wait()
        pltpu.make_async_copy(dst_hbm.at[wid, 0], dst_v.at[q], sem_i[q]).wait()

    def wait_gather(b):
        pltpu.make_async_copy(lin_hbm.at[idx_v.at[0]], rows.at[b],
                              sem_g[b]).wait()

    def wait_scatter(b):
        pltpu.make_async_copy(lin_hbm.at[idx_v.at[0]], rows.at[b],
                              sem_s[b]).wait()

    pltpu.sync_copy(zeros_hbm.at[pl.ds(lo, RPT)], acc_sh.at[pl.ds(lo, RPT)])
    load_idx(0, 0, sync=True)
    load_idx(1, 1, sync=True)
    pltpu.async_copy(lin_hbm.at[idx_v.at[0]], rows.at[0], sem_g[0])
    load_idx(2, 2)
    load_idx(3, 3)
    plsc.subcore_barrier()

    @pl.loop(0, NCHUNK, step=IR)
    def _(g6):
        for u in range(IR):
            g = g6 + u
            b = u % NB
            bn = (u + 1) % NB
            qn = (u + 1) % IR
            qp = (u + 4) % IR

            @pl.when(g >= 2)
            def _retire():
                wait_scatter(bn)

            @pl.when(g + 1 < NCHUNK)
            def _launch_next():
                @pl.when(g + 1 >= 2)
                def _w():
                    wait_idx(qn)
                pltpu.async_copy(lin_hbm.at[idx_v.at[qn]], rows.at[bn],
                                 sem_g[bn])

            wait_gather(b)
            pltpu.async_copy(rows.at[b], acc_sh.at[dst_v.at[u]],
                             sem_s[b], add=True)

            @pl.when(g + 4 < NCHUNK)
            def _prefetch_idx():
                load_idx(g + 4, qp)

    wait_scatter((NCHUNK - 2) % NB)
    wait_scatter((NCHUNK - 1) % NB)
    plsc.subcore_barrier()
    pltpu.sync_copy(acc_sh.at[pl.ds(lo, RPT)], out.at[c, pl.ds(lo, RPT)])


BN = 2048


def _store_lin(lin, lin_ref):
    for t in range(NT):
        lin_ref[t] = lin[:, t * D:(t + 1) * D]


def _lin0_body(h_ref, wcat_ref, bcat_ref, lin_ref):
    _store_lin(
        jnp.dot(h_ref[...], wcat_ref[...], preferred_element_type=jnp.float32)
        + bcat_ref[...], lin_ref)


def _gru_body(h_ref, p0_ref, p1_ref, wih_ref, whh_ref, bih_ref, bhh_ref,
              wcat_ref, bcat_ref, hout_ref, lin_ref):
    h = h_ref[...]
    a = p0_ref[0] + p1_ref[0]
    gi = jnp.dot(a, wih_ref[...], preferred_element_type=jnp.float32) + bih_ref[...]
    gh = jnp.dot(h, whh_ref[...], preferred_element_type=jnp.float32) + bhh_ref[...]
    r = jax.nn.sigmoid(gi[:, :D] + gh[:, :D])
    z = jax.nn.sigmoid(gi[:, D:2 * D] + gh[:, D:2 * D])
    n = jnp.tanh(gi[:, 2 * D:] + r * gh[:, 2 * D:])
    hn = (1.0 - z) * n + z * h
    hout_ref[...] = hn
    if lin_ref is not None:
        _store_lin(
            jnp.dot(hn, wcat_ref[...], preferred_element_type=jnp.float32)
            + bcat_ref[...], lin_ref)


def _final_body(h_ref, p_ref, x_ref, wih_ref, whh_ref, bih_ref, bhh_ref,
                w0, w1, w2, b1, w3, b3, wy, by,
                cw0h, cw0x, cw1h, cw1x, cw2h, cw2x, cb1,
                dw0, dw1, db2, wz, bz, out_ref):
    h = h_ref[...]
    a = p_ref[0, 0:N] + p_ref[1, 0:N]
    gi = jnp.dot(a, wih_ref[...], preferred_element_type=jnp.float32) + bih_ref[...]
    gh = jnp.dot(h, whh_ref[...], preferred_element_type=jnp.float32) + bhh_ref[...]
    r = jax.nn.sigmoid(gi[:, :D] + gh[:, :D])
    z = jax.nn.sigmoid(gi[:, D:2 * D] + gh[:, D:2 * D])
    n = jnp.tanh(gi[:, 2 * D:] + r * gh[:, 2 * D:])
    X = (1.0 - z) * n + z * h

    Y0 = (jnp.dot(X[0:10000], w0[...], preferred_element_type=jnp.float32)
          + jnp.dot(X[1:10001], w1[...], preferred_element_type=jnp.float32)
          + jnp.dot(X[2:10002], w2[...], preferred_element_type=jnp.float32)
          + b1[...])
    Y0 = jnp.maximum(Y0, 0.0)
    Ev = Y0.reshape(5000, 2, D)
    A = jnp.maximum(Ev[:, 0, :], Ev[:, 1, :])
    Y1 = jnp.maximum(A[0:4999], Ev[1:5000, 0, :])
    Y1 = jnp.maximum(jnp.dot(Y1, w3[...], preferred_element_type=jnp.float32)
                     + b3[...], 0.0)
    Yp = Y1[0:4998].reshape(2499, 2, D)
    Y2 = jnp.maximum(Yp[:, 0, :], Yp[:, 1, :])
    yv = jnp.dot(Y2, wy[...], preferred_element_type=jnp.float32) + by[...]

    Xc = x_ref[...]
    Z0 = (jnp.dot(X[0:10000], cw0h[...], preferred_element_type=jnp.float32)
          + jnp.dot(Xc[0:10000], cw0x[...], preferred_element_type=jnp.float32)
          + jnp.dot(X[1:10001], cw1h[...], preferred_element_type=jnp.float32)
          + jnp.dot(Xc[1:10001], cw1x[...], preferred_element_type=jnp.float32)
          + jnp.dot(X[2:10002], cw2h[...], preferred_element_type=jnp.float32)
          + jnp.dot(Xc[2:10002], cw2x[...], preferred_element_type=jnp.float32)
          + cb1[...])
    Z0 = jnp.maximum(Z0, 0.0)
    Ev2 = Z0.reshape(5000, 2, 2 * D)
    A2 = jnp.maximum(Ev2[:, 0, :], Ev2[:, 1, :])
    Z1 = jnp.maximum(A2[0:4999], Ev2[1:5000, 0, :])
    Z2 = (jnp.dot(Z1[0:4998], dw0[...], preferred_element_type=jnp.float32)
          + jnp.dot(Z1[1:4999], dw1[...], preferred_element_type=jnp.float32)
          + db2[...])
    Z2 = jnp.maximum(Z2, 0.0)
    Zp = Z2.reshape(2499, 2, 2 * D)
    Z2p = jnp.maximum(Zp[:, 0, :], Zp[:, 1, :])
    zv = jnp.dot(Z2p, wz[...], preferred_element_type=jnp.float32) + bz[...]
    m = jnp.sum(zv * yv) / jnp.float32(2499.0)
    out_ref[...] = jax.nn.sigmoid(m).reshape(1, 1)


def _row_blocked(shape):
    return pl.BlockSpec((BN,) + shape[1:], lambda i: (i,) + (0,) * (len(shape) - 1))


def _full(shape):
    return pl.BlockSpec(shape, lambda *_: (0,) * len(shape))


def kernel(x, edge_index, etypes, Wt, bt, W_ih, W_hh, b_ih, b_hh,
           conv1_w, conv1_b, conv2_w, conv2_b, convc1_w, convc1_b,
           convc2_w, convc2_b, mlp_y_w, mlp_y_b, mlp_z_w, mlp_z_b):
    f32 = jnp.float32
    h0 = x[0]

    wcat = jnp.transpose(Wt, (2, 0, 1)).reshape(D, NT * D)
    bcat = bt.reshape(1, NT * D)
    wih = W_ih.T
    whh = W_hh.T
    bih = b_ih.reshape(1, 3 * D)
    bhh = b_hh.reshape(1, 3 * D)

    src = edge_index[0]
    dst = edge_index[1]
    gidx = etypes * N8 + src
    ept_real = E // NTILES
    pad = EPT - ept_real
    dummy_g = jnp.broadcast_to((jnp.arange(pad, dtype=jnp.int32) * 97) % (NT * N8),
                               (NTILES, pad))
    dummy_d = jnp.broadcast_to(N + (jnp.arange(pad, dtype=jnp.int32) % (NPAD - N)),
                               (NTILES, pad))
    gidx_p = jnp.concatenate([gidx.reshape(NTILES, ept_real), dummy_g], axis=1)
    dst_p = jnp.concatenate([dst.reshape(NTILES, ept_real), dummy_d], axis=1)
    gidx_p = gidx_p.reshape(NTILES, NCHUNK, CH)
    dst_p = dst_p.reshape(NTILES, NCHUNK, CH)
    zeros_acc = jnp.zeros((NPAD, D), f32)

    grid_n = (N + BN - 1) // BN

    lin_spec = pl.BlockSpec((NT, BN, D), lambda i: (0, i, 0))
    lin_shape = jax.ShapeDtypeStruct((NT, N8, D), f32)
    p0_spec = pl.BlockSpec((1, BN, D), lambda i: (0, i, 0))
    p1_spec = pl.BlockSpec((1, BN, D), lambda i: (1, i, 0))

    lin = pl.pallas_call(
        _lin0_body,
        grid=(grid_n,),
        in_specs=[_row_blocked((N, D)), _full((D, NT * D)), _full((1, NT * D))],
        out_specs=lin_spec,
        out_shape=lin_shape,
    )(h0, wcat, bcat)

    w0 = conv1_w[:, :, 0].T
    w1 = conv1_w[:, :, 1].T
    w2 = conv1_w[:, :, 2].T
    b1 = conv1_b.reshape(1, D)
    w3 = conv2_w[:, :, 0].T
    b3 = conv2_b.reshape(1, D)
    wy = mlp_y_w.T
    by = mlp_y_b.reshape(1, 1)
    C2 = 2 * D
    cw0 = convc1_w[:, :, 0].T
    cw1 = convc1_w[:, :, 1].T
    cw2 = convc1_w[:, :, 2].T
    cb1 = convc1_b.reshape(1, C2)
    dw0 = convc2_w[:, :, 0].T
    dw1 = convc2_w[:, :, 1].T
    db2 = convc2_b.reshape(1, C2)
    wz = mlp_z_w.T
    bz = mlp_z_b.reshape(1, 1)

    h = h0
    for step in range(STEPS - 1):
        lin4 = lin.reshape(NT * N8, D)
        p = _edge_pass(lin4, gidx_p, dst_p, zeros_acc)
        h, lin = pl.pallas_call(
            _gru_body,
            grid=(grid_n,),
            in_specs=[_row_blocked((N, D)), p0_spec, p1_spec,
                      _full((D, 3 * D)),
                      _full((D, 3 * D)), _full((1, 3 * D)),
                      _full((1, 3 * D)), _full((D, NT * D)),
                      _full((1, NT * D))],
            out_specs=[_row_blocked((N, D)), lin_spec],
            out_shape=[jax.ShapeDtypeStruct((N, D), f32), lin_shape],
        )(h, p, p, wih, whh, bih, bhh, wcat, bcat)

    p = _edge_pass(lin.reshape(NT * N8, D), gidx_p, dst_p, zeros_acc)

    out = pl.pallas_call(
        _final_body,
        in_specs=[_full((N, D)), _full((2, NPAD, D)), _full((N, D)),
                  _full((D, 3 * D)), _full((D, 3 * D)), _full((1, 3 * D)),
                  _full((1, 3 * D)),
                  _full((D, D)), _full((D, D)), _full((D, D)), _full((1, D)),
                  _full((D, D)), _full((1, D)), _full((D, 1)), _full((1, 1)),
                  _full((D, C2)), _full((D, C2)), _full((D, C2)),
                  _full((D, C2)), _full((D, C2)), _full((D, C2)),
                  _full((1, C2)), _full((C2, C2)), _full((C2, C2)),
                  _full((1, C2)), _full((C2, 1)), _full((1, 1))],
        out_specs=_full((1, 1)),
        out_shape=jax.ShapeDtypeStruct((1, 1), f32),
    )(h, p, h0, wih, whh, bih, bhh,
      w0, w1, w2, b1, w3, b3, wy, by,
      cw0[:D], cw0[D:], cw1[:D], cw1[D:], cw2[:D], cw2[D:],
      cb1, dw0, dw1, db2, wz, bz)

    return out.reshape(1)


# speedup vs baseline: 1.0050x; 1.0050x over previous
import functools

import jax
import jax.numpy as jnp
from jax import lax
from jax.experimental import pallas as pl
from jax.experimental.pallas import tpu as pltpu
from jax.experimental.pallas import tpu_sc as plsc

N = 10002
N8 = 10008
E = 320064
D = 128
NT = 4
STEPS = 6

NTILES = 32
NPAD = 10112
RPT = NPAD // 16
CH = 112
NCHUNK = 90
EPT = NCHUNK * CH
EPAD = EPT * NTILES
NB = 3
IR = 6

_sc_mesh = plsc.VectorSubcoreMesh(core_axis_name="c", subcore_axis_name="s")


@functools.partial(
    pl.kernel,
    out_type=jax.ShapeDtypeStruct((2, NPAD, D), jnp.float32),
    mesh=_sc_mesh,
    scratch_types=[
        pltpu.VMEM((IR, CH), jnp.int32),
        pltpu.VMEM((IR, CH), jnp.int32),
        pltpu.VMEM((NB, CH, D), jnp.float32),
        pltpu.VMEM_SHARED((NPAD, D), jnp.float32),
    ] + [pltpu.SemaphoreType.DMA] * (2 * NB + IR),
)
def _edge_pass(lin_hbm, gidx_hbm, dst_hbm, zeros_hbm,
               out, idx_v, dst_v, rows, acc_sh, *sems):
    sem_g = sems[:NB]
    sem_s = sems[NB:2 * NB]
    sem_i = sems[2 * NB:]
    c = lax.axis_index("c")
    s = lax.axis_index("s")
    lo = s * RPT
    wid = c * 16 + s

    def load_idx(k, q, sync=False):
        if sync:
            pltpu.sync_copy(gidx_hbm.at[wid, k], idx_v.at[q])
            pltpu.sync_copy(dst_hbm.at[wid, k], dst_v.at[q])
        else:
            pltpu.async_copy(gidx_hbm.at[wid, k], idx_v.at[q], sem_i[q])
            pltpu.async_copy(dst_hbm.at[wid, k], dst_v.at[q], sem_i[q])

    def wait_idx(q):
        pltpu.make_async_copy(gidx_hbm.at[wid, 0], idx_v.at[q], sem_i[q]).wait()
        pltpu.make_async_copy(dst_hbm.at[wid, 0], dst_v.at[q], sem_i[q]).wait()

    def wait_gather(b):
        pltpu.make_async_copy(lin_hbm.at[idx_v.at[0]], rows.at[b],
                              sem_g[b]).wait()

    def wait_scatter(b):
        pltpu.make_async_copy(lin_hbm.at[idx_v.at[0]], rows.at[b],
                              sem_s[b]).wait()

    pltpu.sync_copy(zeros_hbm.at[pl.ds(lo, RPT)], acc_sh.at[pl.ds(lo, RPT)])
    load_idx(0, 0, sync=True)
    load_idx(1, 1, sync=True)
    pltpu.async_copy(lin_hbm.at[idx_v.at[0]], rows.at[0], sem_g[0])
    load_idx(2, 2)
    load_idx(3, 3)
    plsc.subcore_barrier()

    @pl.loop(0, NCHUNK, step=IR)
    def _(g6):
        for u in range(IR):
            g = g6 + u
            b = u % NB
            bn = (u + 1) % NB
            qn = (u + 1) % IR
            qp = (u + 4) % IR

            @pl.when(g >= 2)
            def _retire():
                wait_scatter(bn)

            @pl.when(g + 1 < NCHUNK)
            def _launch_next():
                @pl.when(g + 1 >= 2)
                def _w():
                    wait_idx(qn)
                pltpu.async_copy(lin_hbm.at[idx_v.at[qn]], rows.at[bn],
                                 sem_g[bn])

            wait_gather(b)
            pltpu.async_copy(rows.at[b], acc_sh.at[dst_v.at[u]],
                             sem_s[b], add=True)

            @pl.when(g + 4 < NCHUNK)
            def _prefetch_idx():
                load_idx(g + 4, qp)

    wait_scatter((NCHUNK - 2) % NB)
    wait_scatter((NCHUNK - 1) % NB)
    plsc.subcore_barrier()
    pltpu.sync_copy(acc_sh.at[pl.ds(lo, RPT)], out.at[c, pl.ds(lo, RPT)])


BN = 2048


def _store_lin(lin, lin_ref):
    for t in range(NT):
        lin_ref[t] = lin[:, t * D:(t + 1) * D]


def _lin0_body(h_ref, wcat_ref, bcat_ref, lin_ref):
    _store_lin(
        jnp.dot(h_ref[...], wcat_ref[...], preferred_element_type=jnp.float32)
        + bcat_ref[...], lin_ref)


def _gru_body(h_ref, p0_ref, p1_ref, wih_ref, whh_ref, bih_ref, bhh_ref,
              wcat_ref, bcat_ref, hout_ref, lin_ref):
    h = h_ref[...]
    a = p0_ref[0] + p1_ref[0]
    gi = jnp.dot(a, wih_ref[...], preferred_element_type=jnp.float32) + bih_ref[...]
    gh = jnp.dot(h, whh_ref[...], preferred_element_type=jnp.float32) + bhh_ref[...]
    r = jax.nn.sigmoid(gi[:, :D] + gh[:, :D])
    z = jax.nn.sigmoid(gi[:, D:2 * D] + gh[:, D:2 * D])
    n = jnp.tanh(gi[:, 2 * D:] + r * gh[:, 2 * D:])
    hn = (1.0 - z) * n + z * h
    hout_ref[...] = hn
    if lin_ref is not None:
        _store_lin(
            jnp.dot(hn, wcat_ref[...], preferred_element_type=jnp.float32)
            + bcat_ref[...], lin_ref)


def _final_body(h_ref, p_ref, x_ref, wih_ref, whh_ref, bih_ref, bhh_ref,
                w0, w1, w2, b1, w3, b3, wy, by,
                cw0h, cw0x, cw1h, cw1x, cw2h, cw2x, cb1,
                dw0, dw1, db2, wz, bz, out_ref):
    h = h_ref[...]
    a = p_ref[0, 0:N] + p_ref[1, 0:N]
    gi = jnp.dot(a, wih_ref[...], preferred_element_type=jnp.float32) + bih_ref[...]
    gh = jnp.dot(h, whh_ref[...], preferred_element_type=jnp.float32) + bhh_ref[...]
    r = jax.nn.sigmoid(gi[:, :D] + gh[:, :D])
    z = jax.nn.sigmoid(gi[:, D:2 * D] + gh[:, D:2 * D])
    n = jnp.tanh(gi[:, 2 * D:] + r * gh[:, 2 * D:])
    X = (1.0 - z) * n + z * h

    Y0 = (jnp.dot(X[0:10000], w0[...], preferred_element_type=jnp.float32)
          + jnp.dot(X[1:10001], w1[...], preferred_element_type=jnp.float32)
          + jnp.dot(X[2:10002], w2[...], preferred_element_type=jnp.float32)
          + b1[...])
    Y0 = jnp.maximum(Y0, 0.0)
    Ev = Y0.reshape(5000, 2, D)
    A = jnp.maximum(Ev[:, 0, :], Ev[:, 1, :])
    Y1 = jnp.maximum(A[0:4999], Ev[1:5000, 0, :])
    Y1 = jnp.maximum(jnp.dot(Y1, w3[...], preferred_element_type=jnp.float32)
                     + b3[...], 0.0)
    Yp = Y1[0:4998].reshape(2499, 2, D)
    Y2 = jnp.maximum(Yp[:, 0, :], Yp[:, 1, :])
    yv = jnp.dot(Y2, wy[...], preferred_element_type=jnp.float32) + by[...]

    Xc = x_ref[...]
    Z0 = (jnp.dot(X[0:10000], cw0h[...], preferred_element_type=jnp.float32)
          + jnp.dot(Xc[0:10000], cw0x[...], preferred_element_type=jnp.float32)
          + jnp.dot(X[1:10001], cw1h[...], preferred_element_type=jnp.float32)
          + jnp.dot(Xc[1:10001], cw1x[...], preferred_element_type=jnp.float32)
          + jnp.dot(X[2:10002], cw2h[...], preferred_element_type=jnp.float32)
          + jnp.dot(Xc[2:10002], cw2x[...], preferred_element_type=jnp.float32)
          + cb1[...])
    Z0 = jnp.maximum(Z0, 0.0)
    Ev2 = Z0.reshape(5000, 2, 2 * D)
    A2 = jnp.maximum(Ev2[:, 0, :], Ev2[:, 1, :])
    Z1 = jnp.maximum(A2[0:4999], Ev2[1:5000, 0, :])
    Z2 = (jnp.dot(Z1[0:4998], dw0[...], preferred_element_type=jnp.float32)
          + jnp.dot(Z1[1:4999], dw1[...], preferred_element_type=jnp.float32)
          + db2[...])
    Z2 = jnp.maximum(Z2, 0.0)
    Zp = Z2.reshape(2499, 2, 2 * D)
    Z2p = jnp.maximum(Zp[:, 0, :], Zp[:, 1, :])
    zv = jnp.dot(Z2p, wz[...], preferred_element_type=jnp.float32) + bz[...]
    m = jnp.sum(zv * yv) / jnp.float32(2499.0)
    out_ref[...] = jax.nn.sigmoid(m).reshape(1, 1)


def _row_blocked(shape):
    return pl.BlockSpec((BN,) + shape[1:], lambda i: (i,) + (0,) * (len(shape) - 1))


def _full(shape):
    return pl.BlockSpec(shape, lambda *_: (0,) * len(shape))


def kernel(x, edge_index, etypes, Wt, bt, W_ih, W_hh, b_ih, b_hh,
           conv1_w, conv1_b, conv2_w, conv2_b, convc1_w, convc1_b,
           convc2_w, convc2_b, mlp_y_w, mlp_y_b, mlp_z_w, mlp_z_b):
    f32 = jnp.float32
    h0 = x[0]

    wcat = jnp.transpose(Wt, (2, 0, 1)).reshape(D, NT * D)
    bcat = bt.reshape(1, NT * D)
    wih = W_ih.T
    whh = W_hh.T
    bih = b_ih.reshape(1, 3 * D)
    bhh = b_hh.reshape(1, 3 * D)

    src = edge_index[0]
    dst = edge_index[1]
    gidx = etypes * N8 + src
    ept_real = E // NTILES
    pad = EPT - ept_real
    dummy_g = jnp.broadcast_to((jnp.arange(pad, dtype=jnp.int32) * 97) % (NT * N8),
                               (NTILES, pad))
    dummy_d = jnp.broadcast_to(N + (jnp.arange(pad, dtype=jnp.int32) % (NPAD - N)),
                               (NTILES, pad))
    gidx_p = jnp.concatenate([gidx.reshape(NTILES, ept_real), dummy_g], axis=1)
    dst_p = jnp.concatenate([dst.reshape(NTILES, ept_real), dummy_d], axis=1)
    gidx_p = gidx_p.reshape(NTILES, NCHUNK, CH)
    dst_p = dst_p.reshape(NTILES, NCHUNK, CH)
    zeros_acc = jnp.zeros((NPAD, D), f32)

    grid_n = (N + BN - 1) // BN

    lin_spec = pl.BlockSpec((NT, BN, D), lambda i: (0, i, 0))
    lin_shape = jax.ShapeDtypeStruct((NT, N8, D), f32)
    p0_spec = pl.BlockSpec((1, BN, D), lambda i: (0, i, 0))
    p1_spec = pl.BlockSpec((1, BN, D), lambda i: (1, i, 0))

    lin = pl.pallas_call(
        _lin0_body,
        grid=(grid_n,),
        in_specs=[_row_blocked((N, D)), _full((D, NT * D)), _full((1, NT * D))],
        out_specs=lin_spec,
        out_shape=lin_shape,
    )(h0, wcat, bcat)

    w0 = conv1_w[:, :, 0].T
    w1 = conv1_w[:, :, 1].T
    w2 = conv1_w[:, :, 2].T
    b1 = conv1_b.reshape(1, D)
    w3 = conv2_w[:, :, 0].T
    b3 = conv2_b.reshape(1, D)
    wy = mlp_y_w.T
    by = mlp_y_b.reshape(1, 1)
    C2 = 2 * D
    cw0 = convc1_w[:, :, 0].T
    cw1 = convc1_w[:, :, 1].T
    cw2 = convc1_w[:, :, 2].T
    cb1 = convc1_b.reshape(1, C2)
    dw0 = convc2_w[:, :, 0].T
    dw1 = convc2_w[:, :, 1].T
    db2 = convc2_b.reshape(1, C2)
    wz = mlp_z_w.T
    bz = mlp_z_b.reshape(1, 1)

    h = h0
    for step in range(STEPS - 1):
        lin4 = lin.reshape(NT * N8, D)
        p = _edge_pass(lin4, gidx_p, dst_p, zeros_acc)
        h, lin = pl.pallas_call(
            _gru_body,
            grid=(grid_n,),
            in_specs=[_row_blocked((N, D)), p0_spec, p1_spec,
                      _full((D, 3 * D)),
                      _full((D, 3 * D)), _full((1, 3 * D)),
                      _full((1, 3 * D)), _full((D, NT * D)),
                      _full((1, NT * D))],
            out_specs=[_row_blocked((N, D)), lin_spec],
            out_shape=[jax.ShapeDtypeStruct((N, D), f32), lin_shape],
        )(h, p, p, wih, whh, bih, bhh, wcat, bcat)

    p = _edge_pass(lin.reshape(NT * N8, D), gidx_p, dst_p, zeros_acc)

    out = pl.pallas_call(
        _final_body,
        in_specs=[_full((N, D)), _full((2, NPAD, D)), _full((N, D)),
                  _full((D, 3 * D)), _full((D, 3 * D)), _full((1, 3 * D)),
                  _full((1, 3 * D)),
                  _full((D, D)), _full((D, D)), _full((D, D)), _full((1, D)),
                  _full((D, D)), _full((1, D)), _full((D, 1)), _full((1, 1)),
                  _full((D, C2)), _full((D, C2)), _full((D, C2)),
                  _full((D, C2)), _full((D, C2)), _full((D, C2)),
                  _full((1, C2)), _full((C2, C2)), _full((C2, C2)),
                  _full((1, C2)), _full((C2, 1)), _full((1, 1))],
        out_specs=_full((1, 1)),
        out_shape=jax.ShapeDtypeStruct((1, 1), f32),
    )(h, p, h0, wih, whh, bih, bhh,
      w0, w1, w2, b1, w3, b3, wy, by,
      cw0[:D], cw0[D:], cw1[:D], cw1[D:], cw2[:D], cw2[D:],
      cb1, dw0, dw1, db2, wz, bz)

    return out.reshape(1)
